# Initial kernel scaffold; baseline (speedup 1.0000x reference)
import numpy as np
import jax
import jax.numpy as jnp
from jax import lax
from jax.experimental import pallas as pl
from jax.experimental.pallas import tpu as pltpu
from jax.experimental.pallas import tpu_sc as plsc

N_LEVELS = 16
N_FEATS = 2
LOG2_HASHMAP_SIZE = 19
TABLE_SIZE = 1 << LOG2_HASHMAP_SIZE
BASE_RES = 16.0
FINEST_RES = 512.0
D = 3
N_POINTS = 262144

NC = 2
NS = 16
NW = NC * NS
PTS_PER_W = N_POINTS // NW
C = 1024
NCHUNK = PTS_PER_W // C
G = C // 16

_B = np.exp((np.log(FINEST_RES) - np.log(BASE_RES)) / (N_LEVELS - 1))
_GS = [np.float32(np.float32(2.0) / np.float32(np.floor(BASE_RES * _B**l)))
       for l in range(N_LEVELS)]
_P1 = int(np.uint32(2654435761).astype(np.int32))
_P2 = int(np.uint32(805459861).astype(np.int32))
_MASK = (1 << LOG2_HASHMAP_SIZE) - 1


def _sc_body(xf_hbm, tf_hbm, out_hbm,
             xbuf, wbuf0, wbuf1, idxbuf0, idxbuf1, rowsbuf0, rowsbuf1,
             outbuf, sem0, sem1):
    wid = lax.axis_index("s") * NC + lax.axis_index("c")
    base0 = wid * PTS_PER_W
    iota = lax.iota(jnp.int32, 16)
    col0 = jnp.zeros((16,), jnp.int32)
    col1 = jnp.ones((16,), jnp.int32)
    wbufs = (wbuf0, wbuf1)
    idxbufs = (idxbuf0, idxbuf1)
    rowsbufs = (rowsbuf0, rowsbuf1)
    sems = (sem0, sem1)

    def compute_idx(lvl, wb, idxb):
        gs = _GS[lvl]
        lvl_base = lvl * TABLE_SIZE

        @pl.loop(0, G)
        def _(i):
            s = i * 16
            bls = []
            for d in range(D):
                xd = xbuf[pl.ds(d * C + s, 16)]
                t = (xd + jnp.float32(1.0)) / gs
                bl = t.astype(jnp.int32)
                blf = bl.astype(jnp.float32)
                vmin = blf * gs + jnp.float32(-1.0)
                vmax = vmin + gs
                wb[pl.ds(d * C + s, 16)] = (xd - vmin) / (vmax - vmin)
                bls.append(bl)
            bl0, bl1, bl2 = bls
            m1 = bl1 * jnp.int32(_P1)
            m1b = m1 + jnp.int32(_P1)
            m2 = bl2 * jnp.int32(_P2)
            m2b = m2 + jnp.int32(_P2)
            bl0b = bl0 + jnp.int32(1)
            e = (bl0 ^ m1, bl0 ^ m1b, bl0b ^ m1, bl0b ^ m1b)
            for j in range(8):
                b2, b1, b0 = (j >> 2) & 1, (j >> 1) & 1, j & 1
                h = e[2 * b2 + b1] ^ (m2b if b0 else m2)
                h = (h & jnp.int32(_MASK)) + jnp.int32(lvl_base)
                idxb[pl.ds(j * C + s, 16)] = h

    def interp(lvl, wb, rowsb):
        @pl.loop(0, G)
        def _(i):
            s = i * 16
            w0 = wb[pl.ds(s, 16)]
            w1 = wb[pl.ds(C + s, 16)]
            w2 = wb[pl.ds(2 * C + s, 16)]
            w01 = w0 * w1
            w02 = w0 * w2
            w12 = w1 * w2
            w012 = w01 * w2
            wcs = (None, w2, w1, w12, w0, w02, w01, w012)
            rid0 = iota + s
            acc0 = plsc.load_gather(rowsb, [rid0, col0])
            acc1 = plsc.load_gather(rowsb, [rid0, col1])
            for j in range(1, 8):
                rid = rid0 + j * C
                f0 = plsc.load_gather(rowsb, [rid, col0])
                f1 = plsc.load_gather(rowsb, [rid, col1])
                acc0 = acc0 + f0 * wcs[j]
                acc1 = acc1 + f1 * wcs[j]
            outbuf[2 * lvl, pl.ds(s, 16)] = acc0
            outbuf[2 * lvl + 1, pl.ds(s, 16)] = acc1

    @pl.loop(0, NCHUNK)
    def _(ci):
        base = base0 + ci * C
        for d in range(D):
            pltpu.sync_copy(xf_hbm.at[pl.ds(d * N_POINTS + base, C)],
                            xbuf.at[pl.ds(d * C, C)])
        compute_idx(0, wbufs[0], idxbufs[0])
        copies = [pltpu.async_copy(tf_hbm.at[idxbufs[0]], rowsbufs[0], sems[0])]
        for l in range(N_LEVELS):
            if l + 1 < N_LEVELS:
                p = (l + 1) % 2
                compute_idx(l + 1, wbufs[p], idxbufs[p])
                copies.append(
                    pltpu.async_copy(tf_hbm.at[idxbufs[p]], rowsbufs[p], sems[p]))
            copies[l].wait()
            interp(l, wbufs[l % 2], rowsbufs[l % 2])
        pltpu.sync_copy(outbuf, out_hbm.at[:, pl.ds(base, C)])


@jax.jit
def kernel(x, tables):
    xf = x.T.reshape(D * N_POINTS)
    tf = tables.reshape(N_LEVELS * TABLE_SIZE, N_FEATS)
    mesh = plsc.VectorSubcoreMesh(core_axis_name="c", subcore_axis_name="s",
                                  num_cores=NC, num_subcores=NS)
    kern = pl.kernel(
        _sc_body,
        out_type=jax.ShapeDtypeStruct((2 * N_LEVELS, N_POINTS), jnp.float32),
        mesh=mesh,
        scratch_types=[
            pltpu.VMEM((D * C,), jnp.float32),
            pltpu.VMEM((D * C,), jnp.float32),
            pltpu.VMEM((D * C,), jnp.float32),
            pltpu.VMEM((8 * C,), jnp.int32),
            pltpu.VMEM((8 * C,), jnp.int32),
            pltpu.VMEM((8 * C, N_FEATS), jnp.float32),
            pltpu.VMEM((8 * C, N_FEATS), jnp.float32),
            pltpu.VMEM((2 * N_LEVELS, C), jnp.float32),
            pltpu.SemaphoreType.DMA,
            pltpu.SemaphoreType.DMA,
        ],
    )
    out_t = kern(xf, tf)
    return out_t.T


# speedup vs baseline: 20.0826x; 20.0826x over previous
import numpy as np
import jax
import jax.numpy as jnp
from jax import lax
from jax.experimental import pallas as pl
from jax.experimental.pallas import tpu as pltpu
from jax.experimental.pallas import tpu_sc as plsc

N_LEVELS = 16
N_FEATS = 2
TABLE_SIZE = 1 << 19
BASE_RES = 16.0
FINEST_RES = 512.0
D = 3
N_POINTS = 262144

NC = 2
NS = 16
NW = NC * NS
PTS_PER_W = N_POINTS // NW
C = 1024
NCHUNK = PTS_PER_W // C
G = C // 16

_B = np.exp((np.log(FINEST_RES) - np.log(BASE_RES)) / (N_LEVELS - 1))
_GS = [np.float32(np.float32(2.0) / np.float32(np.floor(BASE_RES * _B**l)))
       for l in range(N_LEVELS)]
_P1 = int(np.uint32(2654435761).astype(np.int32))
_P2 = int(np.uint32(805459861).astype(np.int32))
_MASK = (1 << 19) - 1


def _sc_body(xf_hbm, tf_hbm, out_hbm,
             xbuf, wbuf0, wbuf1, idxbuf0, idxbuf1, rowsbuf0, rowsbuf1,
             outbuf, sem0, sem1):
    wid = lax.axis_index("s") * NC + lax.axis_index("c")
    base0 = wid * PTS_PER_W
    wbufs = (wbuf0, wbuf1)
    idxbufs = (idxbuf0, idxbuf1)
    rowsbufs = (rowsbuf0, rowsbuf1)
    sems = (sem0, sem1)

    def compute_idx(lvl, wb, idxb):
        gs = _GS[lvl]
        word_base = 2 * lvl * TABLE_SIZE

        @pl.loop(0, G)
        def _(i):
            s = i * 16
            bls = []
            for d in range(D):
                xd = xbuf[pl.ds(d * C + s, 16)]
                t = (xd + jnp.float32(1.0)) / gs
                bl = t.astype(jnp.int32)
                blf = bl.astype(jnp.float32)
                vmin = blf * gs + jnp.float32(-1.0)
                vmax = vmin + gs
                wb[pl.ds(d * C + s, 16)] = (xd - vmin) / (vmax - vmin)
                bls.append(bl)
            bl0, bl1, bl2 = bls
            m1 = bl1 * jnp.int32(_P1)
            m1b = m1 + jnp.int32(_P1)
            m2 = bl2 * jnp.int32(_P2)
            m2b = m2 + jnp.int32(_P2)
            bl0b = bl0 + jnp.int32(1)
            e = (bl0 ^ m1, bl0 ^ m1b, bl0b ^ m1, bl0b ^ m1b)
            for j in range(8):
                b2, b1, b0 = (j >> 2) & 1, (j >> 1) & 1, j & 1
                h = e[2 * b2 + b1] ^ (m2b if b0 else m2)
                w0i = ((h & jnp.int32(_MASK)) << 1) + jnp.int32(word_base)
                idxb[pl.ds(j * C + s, 16)] = w0i
                idxb[pl.ds(8 * C + j * C + s, 16)] = w0i + jnp.int32(1)

    def interp(lvl, wb, rowsb):
        @pl.loop(0, G)
        def _(i):
            s = i * 16
            w0 = wb[pl.ds(s, 16)]
            w1 = wb[pl.ds(C + s, 16)]
            w2 = wb[pl.ds(2 * C + s, 16)]
            w01 = w0 * w1
            w02 = w0 * w2
            w12 = w1 * w2
            w012 = w01 * w2
            wcs = (None, w2, w1, w12, w0, w02, w01, w012)
            acc0 = rowsb[pl.ds(s, 16)]
            acc1 = rowsb[pl.ds(8 * C + s, 16)]
            for j in range(1, 8):
                f0 = rowsb[pl.ds(j * C + s, 16)]
                f1 = rowsb[pl.ds(8 * C + j * C + s, 16)]
                acc0 = acc0 + f0 * wcs[j]
                acc1 = acc1 + f1 * wcs[j]
            outbuf[2 * lvl, pl.ds(s, 16)] = acc0
            outbuf[2 * lvl + 1, pl.ds(s, 16)] = acc1

    @pl.loop(0, NCHUNK)
    def _(ci):
        base = base0 + ci * C
        for d in range(D):
            pltpu.sync_copy(xf_hbm.at[pl.ds(d * N_POINTS + base, C)],
                            xbuf.at[pl.ds(d * C, C)])
        compute_idx(0, wbufs[0], idxbufs[0])
        copies = [pltpu.async_copy(tf_hbm.at[idxbufs[0]], rowsbufs[0], sems[0])]
        for l in range(N_LEVELS):
            if l + 1 < N_LEVELS:
                p = (l + 1) % 2
                compute_idx(l + 1, wbufs[p], idxbufs[p])
                copies.append(
                    pltpu.async_copy(tf_hbm.at[idxbufs[p]], rowsbufs[p], sems[p]))
            copies[l].wait()
            interp(l, wbufs[l % 2], rowsbufs[l % 2])
        pltpu.sync_copy(outbuf, out_hbm.at[:, pl.ds(base, C)])


def _build_kernel(interpret=False):
    mesh = plsc.VectorSubcoreMesh(core_axis_name="c", subcore_axis_name="s",
                                  num_cores=NC, num_subcores=NS)
    return pl.kernel(
        _sc_body,
        out_type=jax.ShapeDtypeStruct((2 * N_LEVELS, N_POINTS), jnp.float32),
        mesh=mesh,
        interpret=interpret,
        scratch_types=[
            pltpu.VMEM((D * C,), jnp.float32),
            pltpu.VMEM((D * C,), jnp.float32),
            pltpu.VMEM((D * C,), jnp.float32),
            pltpu.VMEM((2 * 8 * C,), jnp.int32),
            pltpu.VMEM((2 * 8 * C,), jnp.int32),
            pltpu.VMEM((2 * 8 * C,), jnp.float32),
            pltpu.VMEM((2 * 8 * C,), jnp.float32),
            pltpu.VMEM((2 * N_LEVELS, C), jnp.float32),
            pltpu.SemaphoreType.DMA,
            pltpu.SemaphoreType.DMA,
        ],
    )


@jax.jit
def kernel(x, tables):
    xf = x.T.reshape(D * N_POINTS)
    tf = tables.reshape(N_LEVELS * TABLE_SIZE * N_FEATS)
    out_t = _build_kernel()(xf, tf)
    return out_t.T
